# Initial kernel scaffold; baseline (speedup 1.0000x reference)
import jax
import jax.numpy as jnp
from jax.experimental import pallas as pl


def kernel(x, weight):
    raise NotImplementedError("write your pallas kernel here")


# speedup vs baseline: 2.1840x; 2.1840x over previous
import functools

import jax
import jax.numpy as jnp
from jax import lax
from jax.experimental import pallas as pl
from jax.experimental.pallas import tpu as pltpu
from jax.experimental.pallas import tpu_sc as plsc

NUM_POSITIONS = 8192
EMBED_DIM = 1024
B_TOTAL = 4 * 8192

_info = plsc.get_sparse_core_info()
_NC, _NS = _info.num_cores, _info.num_subcores
_NW = _NC * _NS
_B_PER_W = B_TOTAL // _NW
_CHUNK = 64
_N_CHUNKS = _B_PER_W // _CHUNK


def _gather_kernel(x_hbm, w_hbm, out_hbm, idx_v, rows_v, sem):
    wid = lax.axis_index("s") * _NC + lax.axis_index("c")
    base = wid * _B_PER_W
    pltpu.sync_copy(x_hbm.at[pl.ds(base, _B_PER_W)], idx_v)

    def body(i, _):
        off = i * _CHUNK
        pltpu.async_copy(w_hbm.at[idx_v.at[pl.ds(off, _CHUNK)]], rows_v, sem).wait()
        pltpu.sync_copy(rows_v, out_hbm.at[pl.ds(base + off, _CHUNK)])
        return ()

    lax.fori_loop(0, _N_CHUNKS, body, (), unroll=False)


@jax.jit
def kernel(x, weight):
    x_flat = x.reshape(B_TOTAL).astype(jnp.int32)
    mesh = plsc.VectorSubcoreMesh(core_axis_name="c", subcore_axis_name="s")
    out = pl.kernel(
        _gather_kernel,
        mesh=mesh,
        out_type=jax.ShapeDtypeStruct((B_TOTAL, EMBED_DIM), jnp.float32),
        scratch_types=[
            pltpu.VMEM((_B_PER_W,), jnp.int32),
            pltpu.VMEM((_CHUNK, EMBED_DIM), jnp.float32),
            pltpu.SemaphoreType.DMA,
        ],
    )(x_flat, weight)
    return out.reshape(x.shape[0], x.shape[1], EMBED_DIM)


# speedup vs baseline: 2.3016x; 1.0538x over previous
import jax
import jax.numpy as jnp
from jax import lax
from jax.experimental import pallas as pl
from jax.experimental.pallas import tpu as pltpu
from jax.experimental.pallas import tpu_sc as plsc

NUM_POSITIONS = 8192
EMBED_DIM = 1024
B_TOTAL = 4 * 8192

_info = plsc.get_sparse_core_info()
_NC, _NS = _info.num_cores, _info.num_subcores
_NW = _NC * _NS
_B_PER_W = B_TOTAL // _NW
_CHUNK = 32
_N_CHUNKS = _B_PER_W // _CHUNK


def _gather_kernel(x_hbm, w_hbm, out_hbm, idx_v, rows0, rows1, gs0, gs1, ws0, ws1):
    wid = lax.axis_index("s") * _NC + lax.axis_index("c")
    base = wid * _B_PER_W
    pltpu.sync_copy(x_hbm.at[pl.ds(base, _B_PER_W)], idx_v)

    bufs = (rows0, rows1)
    gsems = (gs0, gs1)
    wsems = (ws0, ws1)

    def g_start(i, b):
        pltpu.async_copy(w_hbm.at[idx_v.at[pl.ds(i * _CHUNK, _CHUNK)]],
                         bufs[b], gsems[b])

    def g_wait(b):
        pltpu.make_async_copy(w_hbm.at[pl.ds(0, _CHUNK)], bufs[b],
                              gsems[b]).wait()

    def w_start(i, b):
        pltpu.async_copy(bufs[b], out_hbm.at[pl.ds(base + i * _CHUNK, _CHUNK)],
                         wsems[b])

    def w_wait(b):
        pltpu.make_async_copy(bufs[b], out_hbm.at[pl.ds(base, _CHUNK)],
                              wsems[b]).wait()

    g_start(0, 0)
    g_wait(0)
    w_start(0, 0)
    g_start(1, 1)

    def body(j, _):
        i1 = 1 + 2 * j
        g_wait(1)
        w_start(i1, 1)
        w_wait(0)
        g_start(i1 + 1, 0)
        i2 = 2 + 2 * j
        g_wait(0)
        w_start(i2, 0)
        w_wait(1)
        g_start(i2 + 1, 1)
        return ()

    lax.fori_loop(0, (_N_CHUNKS - 2) // 2, body, (), unroll=False)

    g_wait(1)
    w_start(_N_CHUNKS - 1, 1)
    w_wait(0)
    w_wait(1)


@jax.jit
def kernel(x, weight):
    x_flat = x.reshape(B_TOTAL).astype(jnp.int32)
    mesh = plsc.VectorSubcoreMesh(core_axis_name="c", subcore_axis_name="s")
    out = pl.kernel(
        _gather_kernel,
        mesh=mesh,
        out_type=jax.ShapeDtypeStruct((B_TOTAL, EMBED_DIM), jnp.float32),
        scratch_types=[
            pltpu.VMEM((_B_PER_W,), jnp.int32),
            pltpu.VMEM((_CHUNK, EMBED_DIM), jnp.float32),
            pltpu.VMEM((_CHUNK, EMBED_DIM), jnp.float32),
            pltpu.SemaphoreType.DMA,
            pltpu.SemaphoreType.DMA,
            pltpu.SemaphoreType.DMA,
            pltpu.SemaphoreType.DMA,
        ],
    )(x_flat, weight)
    return out.reshape(x.shape[0], x.shape[1], EMBED_DIM)
